# baseline (speedup 1.0000x reference)
import functools

import jax
import jax.numpy as jnp
from jax import lax
from jax.experimental import pallas as pl
from jax.experimental.pallas import tpu as pltpu
from jax.experimental.pallas import tpu_sc as plsc

VOCAB = 1000000
EMB = 32
NW = 32
NCOLS = VOCAB // 128
TAIL = VOCAB - NCOLS * 128

S_DIM = 4096
T_DIM = 200
NB = S_DIM // 128
G = 8
CHUNK_TOK = G * 128
N_CHUNKS = T_DIM * (NB // G)


def _make_mesh():
    return plsc.VectorSubcoreMesh(core_axis_name="c", subcore_axis_name="s")


@functools.lru_cache(maxsize=None)
def _build_detile():
    n_per_w = NCOLS // NW
    rem = NCOLS - n_per_w * NW

    @functools.partial(
        pl.kernel,
        mesh=_make_mesh(),
        out_type=jax.ShapeDtypeStruct((VOCAB * EMB,), jnp.float32),
        scratch_types=[
            pltpu.VMEM((EMB, 128), jnp.float32),
            pltpu.VMEM((128 * EMB,), jnp.float32),
            pltpu.VMEM((TAIL * EMB,), jnp.float32),
            pltpu.SemaphoreType.DMA,
        ],
        compiler_params=pltpu.CompilerParams(use_tc_tiling_on_sc=True,
                                             needs_layout_passes=False),
    )
    def detile_kernel(wt_hbm, out_hbm, wt_v, lin_v, tail_v, sem):
        wid = lax.axis_index("s") * 2 + lax.axis_index("c")
        iota = lax.iota(jnp.int32, 16)

        def do_col(q):
            pltpu.sync_copy(wt_hbm.at[:, pl.ds(q * 128, 128)], wt_v)
            for l0 in range(8):
                idx = (l0 * 16 + iota) * EMB
                for c in range(EMB):
                    vec = wt_v[c, pl.ds(l0 * 16, 16)]
                    plsc.store_scatter(lin_v, [idx + c], vec)
            pltpu.sync_copy(lin_v, out_hbm.at[pl.ds(q * 128 * EMB, 128 * EMB)])

        def body(i, carry):
            do_col(wid * n_per_w + i)
            return carry

        lax.fori_loop(0, n_per_w, body, 0)

        @pl.when(wid < rem)
        def _():
            do_col(NW * n_per_w + wid)

        @pl.when(wid == rem)
        def _():
            base = NCOLS * 128
            for c in range(EMB):
                pltpu.sync_copy(wt_hbm.at[c, pl.ds(base, TAIL)],
                                wt_v.at[c, pl.ds(0, TAIL)])
            for l0 in range(TAIL // 16):
                idx = (l0 * 16 + iota) * EMB
                for c in range(EMB):
                    vec = wt_v[c, pl.ds(l0 * 16, 16)]
                    plsc.store_scatter(tail_v, [idx + c], vec)
            pltpu.sync_copy(tail_v, out_hbm.at[pl.ds(base * EMB, TAIL * EMB)])

    return detile_kernel


@functools.lru_cache(maxsize=None)
def _build_gather():
    n_per_w = N_CHUNKS // NW
    chunks_per_t = NB // G

    @functools.partial(
        pl.kernel,
        mesh=_make_mesh(),
        out_type=jax.ShapeDtypeStruct((S_DIM * T_DIM * EMB,), jnp.float32),
        scratch_types=[
            pltpu.VMEM((CHUNK_TOK,), jnp.int32),
            pltpu.VMEM((CHUNK_TOK, EMB), jnp.float32),
            pltpu.VMEM((4, G * 1024), jnp.float32),
            pltpu.SemaphoreType.DMA,
            pltpu.SemaphoreType.DMA,
        ],
        compiler_params=pltpu.CompilerParams(use_tc_tiling_on_sc=False,
                                             needs_layout_passes=False),
    )
    def gather_kernel(table_hbm, idx_hbm, out_hbm, idx_v, rows_v, tr_v,
                      sem_g, sem_o):
        wid = lax.axis_index("s") * 2 + lax.axis_index("c")
        iota = lax.iota(jnp.int32, 16)

        def do_chunk(k, carry):
            t = k // chunks_per_t
            b0 = (k % chunks_per_t) * G
            src = t * S_DIM + b0 * 128
            pltpu.sync_copy(idx_hbm.at[pl.ds(src, CHUNK_TOK)], idx_v)
            pltpu.async_copy(table_hbm.at[idx_v], rows_v, sem_g).wait()

            def tr_body(i, carry2):
                bg = i // 8
                l0 = i - bg * 8
                row_idx = bg * 128 + l0 * 16 + iota
                dst_base = bg * 1024 + l0 * 16
                for a in range(4):
                    for c8 in range(8):
                        vec = plsc.load_gather(
                            rows_v, [row_idx, jnp.full((16,), a * 8 + c8,
                                                       jnp.int32)])
                        tr_v[a, pl.ds(dst_base + c8 * 128, 16)] = vec
                return carry2

            lax.fori_loop(0, G * 8, tr_body, 0)
            for a in range(4):
                pltpu.async_copy(
                    tr_v.at[a],
                    out_hbm.at[pl.ds(((t * 4 + a) * NB + b0) * 1024,
                                     G * 1024)],
                    sem_o).wait()
            return carry

        lax.fori_loop(wid * n_per_w, (wid + 1) * n_per_w, do_chunk, 0)

    return gather_kernel


def kernel(tokens, W):
    w_lin = _build_detile()(W.T)
    tf = tokens.T.reshape(S_DIM * T_DIM).astype(jnp.int32)
    flat = _build_gather()(w_lin.reshape(VOCAB, EMB), tf)
    out5 = flat.reshape(T_DIM, 4, NB, 8, 128)
    return out5.transpose(2, 4, 0, 1, 3).reshape(S_DIM, T_DIM, EMB)


# speedup vs baseline: 1.2079x; 1.2079x over previous
import functools

import jax
import jax.numpy as jnp
from jax import lax
from jax.experimental import pallas as pl
from jax.experimental.pallas import tpu as pltpu
from jax.experimental.pallas import tpu_sc as plsc

VOCAB = 1000000
EMB = 32
NW = 32
NCOLS = VOCAB // 128
TAIL = VOCAB - NCOLS * 128

S_DIM = 4096
T_DIM = 200
NB = S_DIM // 128

B_COL = 4
CW = B_COL * 128
N_STEPS = (NCOLS // NW) // B_COL

G = 4
CHUNK_TOK = G * 128
CH_PER_W = T_DIM * (NB // G) // NW


def _make_mesh():
    return plsc.VectorSubcoreMesh(core_axis_name="c", subcore_axis_name="s")


@functools.lru_cache(maxsize=None)
def _build_detile():
    n_cols_w = N_STEPS * B_COL
    rem = NCOLS - n_cols_w * NW

    @functools.partial(
        pl.kernel,
        mesh=_make_mesh(),
        out_type=jax.ShapeDtypeStruct((VOCAB * EMB,), jnp.float32),
        scratch_types=[
            pltpu.VMEM((EMB, CW), jnp.float32),
            pltpu.VMEM((EMB, CW), jnp.float32),
            pltpu.VMEM((CW * EMB,), jnp.float32),
            pltpu.VMEM((CW * EMB,), jnp.float32),
            pltpu.VMEM((TAIL * EMB,), jnp.float32),
            pltpu.SemaphoreType.DMA,
            pltpu.SemaphoreType.DMA,
            pltpu.SemaphoreType.DMA,
            pltpu.SemaphoreType.DMA,
        ],
        compiler_params=pltpu.CompilerParams(use_tc_tiling_on_sc=True,
                                             needs_layout_passes=False),
    )
    def detile_kernel(wt_hbm, out_hbm, wt0, wt1, lin0, lin1, tail_v,
                      si0, si1, so0, so1):
        wid = lax.axis_index("s") * 2 + lax.axis_index("c")
        col0 = wid * n_cols_w
        wt = (wt0, wt1)
        lin = (lin0, lin1)
        si = (si0, si1)
        so = (so0, so1)
        iota = lax.iota(jnp.int32, 16)

        def start_in(step, u):
            q = col0 + step * B_COL
            pltpu.async_copy(wt_hbm.at[:, pl.ds(q * 128, CW)], wt[u], si[u])

        def wait_in(u):
            pltpu.make_async_copy(wt_hbm.at[:, pl.ds(0, CW)], wt[u],
                                  si[u]).wait()

        def start_out(step, u):
            q = col0 + step * B_COL
            pltpu.async_copy(lin[u],
                             out_hbm.at[pl.ds(q * 128 * EMB, CW * EMB)],
                             so[u])

        def wait_out(u):
            pltpu.make_async_copy(lin[u],
                                  out_hbm.at[pl.ds(0, CW * EMB)],
                                  so[u]).wait()

        def transpose(u):
            def tbody(l0, carry):
                base = (l0 * 16 + iota) * EMB
                for c in range(EMB):
                    vec = wt[u][c, pl.ds(l0 * 16, 16)]
                    plsc.store_scatter(lin[u], [base + c], vec)
                return carry
            lax.fori_loop(0, CW // 16, tbody, 0, unroll=2)

        start_in(0, 0)
        start_in(1, 1)

        def body(j, carry):
            for u in (0, 1):
                step = 2 * j + u
                wait_in(u)

                @pl.when(j >= 1)
                def _():
                    wait_out(u)

                transpose(u)
                start_out(step, u)

                @pl.when(step + 2 < N_STEPS)
                def _():
                    start_in(step + 2, u)
            return carry

        lax.fori_loop(0, N_STEPS // 2, body, 0)
        wait_in(0)
        wait_out(0)
        transpose(0)
        start_out(N_STEPS - 1, 0)
        wait_out(1)
        wait_out(0)

        @pl.when(wid < rem)
        def _():
            q = NW * n_cols_w + wid
            pltpu.sync_copy(wt_hbm.at[:, pl.ds(q * 128, 128)],
                            wt0.at[:, pl.ds(0, 128)])

            def tbody(l0, carry):
                base = (l0 * 16 + iota) * EMB
                for c in range(EMB):
                    vec = wt0[c, pl.ds(l0 * 16, 16)]
                    plsc.store_scatter(lin0, [base + c], vec)
                return carry
            lax.fori_loop(0, 8, tbody, 0)
            pltpu.sync_copy(lin0.at[pl.ds(0, 128 * EMB)],
                            out_hbm.at[pl.ds(q * 128 * EMB, 128 * EMB)])

        @pl.when(wid == rem)
        def _():
            base = NCOLS * 128
            for c in range(EMB):
                pltpu.sync_copy(wt_hbm.at[c, pl.ds(base, TAIL)],
                                wt0.at[c, pl.ds(0, TAIL)])
            for l0 in range(TAIL // 16):
                idx = (l0 * 16 + iota) * EMB
                for c in range(EMB):
                    vec = wt0[c, pl.ds(l0 * 16, 16)]
                    plsc.store_scatter(tail_v, [idx + c], vec)
            pltpu.sync_copy(tail_v, out_hbm.at[pl.ds(base * EMB, TAIL * EMB)])

    return detile_kernel


@functools.lru_cache(maxsize=None)
def _build_gather():
    chunks_per_t = NB // G

    @functools.partial(
        pl.kernel,
        mesh=_make_mesh(),
        out_type=jax.ShapeDtypeStruct((S_DIM * T_DIM * EMB,), jnp.float32),
        scratch_types=[
            pltpu.VMEM((CHUNK_TOK,), jnp.int32),
            pltpu.VMEM((CHUNK_TOK,), jnp.int32),
            pltpu.VMEM((CHUNK_TOK, EMB), jnp.float32),
            pltpu.VMEM((CHUNK_TOK, EMB), jnp.float32),
            pltpu.VMEM((4, G * 1024), jnp.float32),
            pltpu.VMEM((4, G * 1024), jnp.float32),
            pltpu.SemaphoreType.DMA,
            pltpu.SemaphoreType.DMA,
            pltpu.SemaphoreType.DMA,
            pltpu.SemaphoreType.DMA,
            pltpu.SemaphoreType.DMA,
            pltpu.SemaphoreType.DMA,
        ],
        compiler_params=pltpu.CompilerParams(use_tc_tiling_on_sc=False,
                                             needs_layout_passes=False),
    )
    def gather_kernel(table_hbm, idx_hbm, out_hbm,
                      idx0, idx1, rows0, rows1, tr0, tr1,
                      smi0, smi1, smg0, smg1, smo0, smo1):
        wid = lax.axis_index("s") * 2 + lax.axis_index("c")
        k0 = wid * CH_PER_W
        idx = (idx0, idx1)
        rows = (rows0, rows1)
        tr = (tr0, tr1)
        smi = (smi0, smi1)
        smg = (smg0, smg1)
        smo = (smo0, smo1)
        iota = lax.iota(jnp.int32, 16)

        def src_off(s):
            k = k0 + s
            t = k // chunks_per_t
            b0 = (k - t * chunks_per_t) * G
            return t, b0, t * S_DIM + b0 * 128

        def start_idx(s, u):
            _, _, off = src_off(s)
            pltpu.async_copy(idx_hbm.at[pl.ds(off, CHUNK_TOK)], idx[u],
                             smi[u])

        def wait_idx(u):
            pltpu.make_async_copy(idx_hbm.at[pl.ds(0, CHUNK_TOK)], idx[u],
                                  smi[u]).wait()

        def start_gather(u):
            pltpu.async_copy(table_hbm.at[idx[u]], rows[u], smg[u])

        def wait_gather(u):
            pltpu.make_async_copy(table_hbm.at[idx[u]], rows[u],
                                  smg[u]).wait()

        def start_out(s, u):
            t, b0, _ = src_off(s)
            for a in range(4):
                pltpu.async_copy(
                    tr[u].at[a],
                    out_hbm.at[pl.ds(((t * 4 + a) * NB + b0) * 1024,
                                     G * 1024)],
                    smo[u])

        def wait_out(u):
            for a in range(4):
                pltpu.make_async_copy(tr[u].at[a],
                                      out_hbm.at[pl.ds(0, G * 1024)],
                                      smo[u]).wait()

        def transpose(u):
            def tbody(i, carry):
                bg = i // 8
                l0 = i - bg * 8
                row_idx = bg * 128 + l0 * 16 + iota
                dst_base = bg * 1024 + l0 * 16
                for a in range(4):
                    for c8 in range(8):
                        vec = plsc.load_gather(
                            rows[u],
                            [row_idx, jnp.full((16,), a * 8 + c8, jnp.int32)])
                        tr[u][a, pl.ds(dst_base + c8 * 128, 16)] = vec
                return carry
            lax.fori_loop(0, G * 8, tbody, 0, unroll=2)

        start_idx(0, 0)
        start_idx(1, 1)
        wait_idx(0)
        start_gather(0)

        def body(j, carry):
            for u in (0, 1):
                s = 2 * j + u
                wait_gather(u)

                @pl.when(j >= 1)
                def _():
                    wait_out(u)

                transpose(u)
                start_out(s, u)

                @pl.when(s + 2 < CH_PER_W)
                def _():
                    start_idx(s + 2, u)

                @pl.when(s + 1 < CH_PER_W)
                def _():
                    wait_idx(1 - u)
                    start_gather(1 - u)
            return carry

        lax.fori_loop(0, CH_PER_W // 2, body, 0)
        wait_out(0)
        wait_out(1)

    return gather_kernel


def kernel(tokens, W):
    w_lin = _build_detile()(W.T)
    tf = tokens.T.reshape(S_DIM * T_DIM).astype(jnp.int32)
    flat = _build_gather()(w_lin.reshape(VOCAB, EMB), tf)
    out5 = flat.reshape(T_DIM, 4, NB, 8, 128)
    return out5.transpose(2, 4, 0, 1, 3).reshape(S_DIM, T_DIM, EMB)


# speedup vs baseline: 1.7325x; 1.4343x over previous
import functools

import jax
import jax.numpy as jnp
from jax import lax
from jax.experimental import pallas as pl
from jax.experimental.pallas import tpu as pltpu
from jax.experimental.pallas import tpu_sc as plsc

VOCAB = 1000000
EMB = 32
NW = 32
NCOLS = VOCAB // 128
TAIL = VOCAB - NCOLS * 128

S_DIM = 4096
T_DIM = 200
NB = S_DIM // 128

B_COL = 4
CW = B_COL * 128
N_STEPS = (NCOLS // NW) // B_COL

G = 4
CHUNK_TOK = G * 128
CH_PER_W = T_DIM * (NB // G) // NW


def _make_mesh():
    return plsc.VectorSubcoreMesh(core_axis_name="c", subcore_axis_name="s")


@functools.lru_cache(maxsize=None)
def _build_detile():
    n_cols_w = N_STEPS * B_COL
    rem = NCOLS - n_cols_w * NW

    @functools.partial(
        pl.kernel,
        mesh=_make_mesh(),
        out_type=jax.ShapeDtypeStruct((VOCAB * EMB,), jnp.float32),
        scratch_types=[
            pltpu.VMEM((EMB, CW), jnp.float32),
            pltpu.VMEM((EMB, CW), jnp.float32),
            pltpu.VMEM((CW * EMB,), jnp.float32),
            pltpu.VMEM((CW * EMB,), jnp.float32),
            pltpu.VMEM((TAIL * EMB,), jnp.float32),
            pltpu.SemaphoreType.DMA,
            pltpu.SemaphoreType.DMA,
            pltpu.SemaphoreType.DMA,
            pltpu.SemaphoreType.DMA,
        ],
        compiler_params=pltpu.CompilerParams(use_tc_tiling_on_sc=True,
                                             needs_layout_passes=False),
    )
    def detile_kernel(wt_hbm, out_hbm, wt0, wt1, lin0, lin1, tail_v,
                      si0, si1, so0, so1):
        wid = lax.axis_index("s") * 2 + lax.axis_index("c")
        col0 = wid * n_cols_w
        wt = (wt0, wt1)
        lin = (lin0, lin1)
        si = (si0, si1)
        so = (so0, so1)
        iota = lax.iota(jnp.int32, 16)

        def start_in(step, u):
            q = col0 + step * B_COL
            pltpu.async_copy(wt_hbm.at[:, pl.ds(q * 128, CW)], wt[u], si[u])

        def wait_in(u):
            pltpu.make_async_copy(wt_hbm.at[:, pl.ds(0, CW)], wt[u],
                                  si[u]).wait()

        def start_out(step, u):
            q = col0 + step * B_COL
            pltpu.async_copy(lin[u],
                             out_hbm.at[pl.ds(q * 128 * EMB, CW * EMB)],
                             so[u])

        def wait_out(u):
            pltpu.make_async_copy(lin[u],
                                  out_hbm.at[pl.ds(0, CW * EMB)],
                                  so[u]).wait()

        def transpose(u):
            def tbody(l0, carry):
                vecs = [wt[u][c, pl.ds(l0 * 16, 16)] for c in range(EMB)]
                base = (l0 * 16 + iota) * EMB
                for c in range(EMB):
                    plsc.store_scatter(lin[u], [base + c], vecs[c])
                return carry
            lax.fori_loop(0, CW // 16, tbody, 0, unroll=2)

        start_in(0, 0)
        start_in(1, 1)

        def body(j, carry):
            for u in (0, 1):
                step = 2 * j + u
                wait_in(u)

                @pl.when(j >= 1)
                def _():
                    wait_out(u)

                transpose(u)
                start_out(step, u)

                @pl.when(step + 2 < N_STEPS)
                def _():
                    start_in(step + 2, u)
            return carry

        lax.fori_loop(0, N_STEPS // 2, body, 0)
        wait_in(0)
        wait_out(0)
        transpose(0)
        start_out(N_STEPS - 1, 0)
        wait_out(1)
        wait_out(0)

        @pl.when(wid < rem)
        def _():
            q = NW * n_cols_w + wid
            pltpu.sync_copy(wt_hbm.at[:, pl.ds(q * 128, 128)],
                            wt0.at[:, pl.ds(0, 128)])

            def tbody(l0, carry):
                base = (l0 * 16 + iota) * EMB
                for c in range(EMB):
                    vec = wt0[c, pl.ds(l0 * 16, 16)]
                    plsc.store_scatter(lin0, [base + c], vec)
                return carry
            lax.fori_loop(0, 8, tbody, 0)
            pltpu.sync_copy(lin0.at[pl.ds(0, 128 * EMB)],
                            out_hbm.at[pl.ds(q * 128 * EMB, 128 * EMB)])

        @pl.when(wid == rem)
        def _():
            base = NCOLS * 128
            for c in range(EMB):
                pltpu.sync_copy(wt_hbm.at[c, pl.ds(base, TAIL)],
                                wt0.at[c, pl.ds(0, TAIL)])
            for l0 in range(TAIL // 16):
                idx = (l0 * 16 + iota) * EMB
                for c in range(EMB):
                    vec = wt0[c, pl.ds(l0 * 16, 16)]
                    plsc.store_scatter(tail_v, [idx + c], vec)
            pltpu.sync_copy(tail_v, out_hbm.at[pl.ds(base * EMB, TAIL * EMB)])

    return detile_kernel


@functools.lru_cache(maxsize=None)
def _build_gather():
    chunks_per_t = NB // G

    @functools.partial(
        pl.kernel,
        mesh=_make_mesh(),
        out_type=jax.ShapeDtypeStruct((S_DIM * T_DIM * EMB,), jnp.float32),
        scratch_types=[
            pltpu.VMEM((CHUNK_TOK,), jnp.int32),
            pltpu.VMEM((CHUNK_TOK,), jnp.int32),
            pltpu.VMEM((CHUNK_TOK, EMB), jnp.float32),
            pltpu.VMEM((CHUNK_TOK, EMB), jnp.float32),
            pltpu.VMEM((4, G * 1024), jnp.float32),
            pltpu.VMEM((4, G * 1024), jnp.float32),
            pltpu.SemaphoreType.DMA,
            pltpu.SemaphoreType.DMA,
            pltpu.SemaphoreType.DMA,
            pltpu.SemaphoreType.DMA,
            pltpu.SemaphoreType.DMA,
            pltpu.SemaphoreType.DMA,
        ],
        compiler_params=pltpu.CompilerParams(use_tc_tiling_on_sc=False,
                                             needs_layout_passes=False),
    )
    def gather_kernel(table_hbm, idx_hbm, out_hbm,
                      idx0, idx1, rows0, rows1, tr0, tr1,
                      smi0, smi1, smg0, smg1, smo0, smo1):
        wid = lax.axis_index("s") * 2 + lax.axis_index("c")
        k0 = wid * CH_PER_W
        idx = (idx0, idx1)
        rows = (rows0, rows1)
        tr = (tr0, tr1)
        smi = (smi0, smi1)
        smg = (smg0, smg1)
        smo = (smo0, smo1)
        iota = lax.iota(jnp.int32, 16)

        def src_off(s):
            k = k0 + s
            t = k // chunks_per_t
            b0 = (k - t * chunks_per_t) * G
            return t, b0, t * S_DIM + b0 * 128

        def start_idx(s, u):
            _, _, off = src_off(s)
            pltpu.async_copy(idx_hbm.at[pl.ds(off, CHUNK_TOK)], idx[u],
                             smi[u])

        def wait_idx(u):
            pltpu.make_async_copy(idx_hbm.at[pl.ds(0, CHUNK_TOK)], idx[u],
                                  smi[u]).wait()

        def start_gather(u):
            pltpu.async_copy(table_hbm.at[idx[u]], rows[u], smg[u])

        def wait_gather(u):
            pltpu.make_async_copy(table_hbm.at[idx[u]], rows[u],
                                  smg[u]).wait()

        def start_out(s, u):
            t, b0, _ = src_off(s)
            for a in range(4):
                pltpu.async_copy(
                    tr[u].at[a],
                    out_hbm.at[pl.ds(((t * 4 + a) * NB + b0) * 1024,
                                     G * 1024)],
                    smo[u])

        def wait_out(u):
            for a in range(4):
                pltpu.make_async_copy(tr[u].at[a],
                                      out_hbm.at[pl.ds(0, G * 1024)],
                                      smo[u]).wait()

        cols = [jnp.full((16,), c, jnp.int32) for c in range(EMB)]

        def transpose(u):
            def tbody(i, carry):
                bg = i // 8
                l0 = i - bg * 8
                row_idx = bg * 128 + l0 * 16 + iota
                dst_base = bg * 1024 + l0 * 16
                vecs = [plsc.load_gather(rows[u], [row_idx, cols[c]])
                        for c in range(EMB)]
                for a in range(4):
                    for c8 in range(8):
                        tr[u][a, pl.ds(dst_base + c8 * 128, 16)] = \
                            vecs[a * 8 + c8]
                return carry
            lax.fori_loop(0, G * 8, tbody, 0, unroll=2)

        start_idx(0, 0)
        start_idx(1, 1)
        wait_idx(0)
        start_gather(0)

        def body(j, carry):
            for u in (0, 1):
                s = 2 * j + u
                wait_gather(u)

                @pl.when(j >= 1)
                def _():
                    wait_out(u)

                transpose(u)
                start_out(s, u)

                @pl.when(s + 2 < CH_PER_W)
                def _():
                    start_idx(s + 2, u)

                @pl.when(s + 1 < CH_PER_W)
                def _():
                    wait_idx(1 - u)
                    start_gather(1 - u)
            return carry

        lax.fori_loop(0, CH_PER_W // 2, body, 0)
        wait_out(0)
        wait_out(1)

    return gather_kernel


def kernel(tokens, W):
    w_lin = _build_detile()(W.T)
    tf = tokens.T.reshape(S_DIM * T_DIM).astype(jnp.int32)
    flat = _build_gather()(w_lin.reshape(VOCAB, EMB), tf)
    out5 = flat.reshape(T_DIM, 4, NB, 8, 128)
    return out5.transpose(2, 4, 0, 1, 3).reshape(S_DIM, T_DIM, EMB)


# speedup vs baseline: 1.7405x; 1.0046x over previous
import functools

import jax
import jax.numpy as jnp
from jax import lax
from jax.experimental import pallas as pl
from jax.experimental.pallas import tpu as pltpu
from jax.experimental.pallas import tpu_sc as plsc

VOCAB = 1000000
EMB = 32
NW = 32
NCOLS = VOCAB // 128
TAIL = VOCAB - NCOLS * 128

S_DIM = 4096
T_DIM = 200
NB = S_DIM // 128

B_COL = 4
CW = B_COL * 128
N_STEPS = (NCOLS // NW) // B_COL

G = 4
CHUNK_TOK = G * 128
CH_PER_W = T_DIM * (NB // G) // NW


def _make_mesh():
    return plsc.VectorSubcoreMesh(core_axis_name="c", subcore_axis_name="s")


@functools.lru_cache(maxsize=None)
def _build_detile():
    n_cols_w = N_STEPS * B_COL
    rem = NCOLS - n_cols_w * NW

    @functools.partial(
        pl.kernel,
        mesh=_make_mesh(),
        out_type=jax.ShapeDtypeStruct((VOCAB * EMB,), jnp.float32),
        scratch_types=[
            pltpu.VMEM((EMB, CW), jnp.float32),
            pltpu.VMEM((EMB, CW), jnp.float32),
            pltpu.VMEM((CW * EMB,), jnp.float32),
            pltpu.VMEM((CW * EMB,), jnp.float32),
            pltpu.VMEM((TAIL * EMB,), jnp.float32),
            pltpu.SemaphoreType.DMA,
            pltpu.SemaphoreType.DMA,
            pltpu.SemaphoreType.DMA,
            pltpu.SemaphoreType.DMA,
        ],
        compiler_params=pltpu.CompilerParams(use_tc_tiling_on_sc=True,
                                             needs_layout_passes=False),
    )
    def detile_kernel(wt_hbm, out_hbm, wt0, wt1, lin0, lin1, tail_v,
                      si0, si1, so0, so1):
        wid = lax.axis_index("s") * 2 + lax.axis_index("c")
        col0 = wid * n_cols_w
        wt = (wt0, wt1)
        lin = (lin0, lin1)
        si = (si0, si1)
        so = (so0, so1)
        iota = lax.iota(jnp.int32, 16)

        def start_in(step, u):
            q = col0 + step * B_COL
            pltpu.async_copy(wt_hbm.at[:, pl.ds(q * 128, CW)], wt[u], si[u])

        def wait_in(u):
            pltpu.make_async_copy(wt_hbm.at[:, pl.ds(0, CW)], wt[u],
                                  si[u]).wait()

        def start_out(step, u):
            q = col0 + step * B_COL
            pltpu.async_copy(lin[u],
                             out_hbm.at[pl.ds(q * 128 * EMB, CW * EMB)],
                             so[u])

        def wait_out(u):
            pltpu.make_async_copy(lin[u],
                                  out_hbm.at[pl.ds(0, CW * EMB)],
                                  so[u]).wait()

        def transpose(u):
            @plsc.parallel_loop(0, CW // 16, unroll=2)
            def tbody(l0):
                vecs = [wt[u][c, pl.ds(l0 * 16, 16)] for c in range(EMB)]
                base = (l0 * 16 + iota) * EMB
                for c in range(EMB):
                    plsc.store_scatter(lin[u], [base + c], vecs[c])

        start_in(0, 0)
        start_in(1, 1)

        def body(j, carry):
            for u in (0, 1):
                step = 2 * j + u
                wait_in(u)

                @pl.when(j >= 1)
                def _():
                    wait_out(u)

                transpose(u)
                start_out(step, u)

                @pl.when(step + 2 < N_STEPS)
                def _():
                    start_in(step + 2, u)
            return carry

        lax.fori_loop(0, N_STEPS // 2, body, 0)
        wait_in(0)
        wait_out(0)
        transpose(0)
        start_out(N_STEPS - 1, 0)
        wait_out(1)
        wait_out(0)

        @pl.when(wid < rem)
        def _():
            q = NW * n_cols_w + wid
            pltpu.sync_copy(wt_hbm.at[:, pl.ds(q * 128, 128)],
                            wt0.at[:, pl.ds(0, 128)])

            def tbody(l0, carry):
                base = (l0 * 16 + iota) * EMB
                for c in range(EMB):
                    vec = wt0[c, pl.ds(l0 * 16, 16)]
                    plsc.store_scatter(lin0, [base + c], vec)
                return carry
            lax.fori_loop(0, 8, tbody, 0)
            pltpu.sync_copy(lin0.at[pl.ds(0, 128 * EMB)],
                            out_hbm.at[pl.ds(q * 128 * EMB, 128 * EMB)])

        @pl.when(wid == rem)
        def _():
            base = NCOLS * 128
            for c in range(EMB):
                pltpu.sync_copy(wt_hbm.at[c, pl.ds(base, TAIL)],
                                wt0.at[c, pl.ds(0, TAIL)])
            for l0 in range(TAIL // 16):
                idx = (l0 * 16 + iota) * EMB
                for c in range(EMB):
                    vec = wt0[c, pl.ds(l0 * 16, 16)]
                    plsc.store_scatter(tail_v, [idx + c], vec)
            pltpu.sync_copy(tail_v, out_hbm.at[pl.ds(base * EMB, TAIL * EMB)])

    return detile_kernel


@functools.lru_cache(maxsize=None)
def _build_gather():
    chunks_per_t = NB // G

    @functools.partial(
        pl.kernel,
        mesh=_make_mesh(),
        out_type=jax.ShapeDtypeStruct((S_DIM * T_DIM * EMB,), jnp.float32),
        scratch_types=[
            pltpu.VMEM((CHUNK_TOK,), jnp.int32),
            pltpu.VMEM((CHUNK_TOK,), jnp.int32),
            pltpu.VMEM((CHUNK_TOK, EMB), jnp.float32),
            pltpu.VMEM((CHUNK_TOK, EMB), jnp.float32),
            pltpu.VMEM((4, G * 1024), jnp.float32),
            pltpu.VMEM((4, G * 1024), jnp.float32),
            pltpu.SemaphoreType.DMA,
            pltpu.SemaphoreType.DMA,
            pltpu.SemaphoreType.DMA,
            pltpu.SemaphoreType.DMA,
            pltpu.SemaphoreType.DMA,
            pltpu.SemaphoreType.DMA,
        ],
        compiler_params=pltpu.CompilerParams(use_tc_tiling_on_sc=False,
                                             needs_layout_passes=False),
    )
    def gather_kernel(table_hbm, idx_hbm, out_hbm,
                      idx0, idx1, rows0, rows1, tr0, tr1,
                      smi0, smi1, smg0, smg1, smo0, smo1):
        wid = lax.axis_index("s") * 2 + lax.axis_index("c")
        k0 = wid * CH_PER_W
        idx = (idx0, idx1)
        rows = (rows0, rows1)
        tr = (tr0, tr1)
        smi = (smi0, smi1)
        smg = (smg0, smg1)
        smo = (smo0, smo1)
        iota = lax.iota(jnp.int32, 16)

        def src_off(s):
            k = k0 + s
            t = k // chunks_per_t
            b0 = (k - t * chunks_per_t) * G
            return t, b0, t * S_DIM + b0 * 128

        def start_idx(s, u):
            _, _, off = src_off(s)
            pltpu.async_copy(idx_hbm.at[pl.ds(off, CHUNK_TOK)], idx[u],
                             smi[u])

        def wait_idx(u):
            pltpu.make_async_copy(idx_hbm.at[pl.ds(0, CHUNK_TOK)], idx[u],
                                  smi[u]).wait()

        def start_gather(u):
            pltpu.async_copy(table_hbm.at[idx[u]], rows[u], smg[u])

        def wait_gather(u):
            pltpu.make_async_copy(table_hbm.at[idx[u]], rows[u],
                                  smg[u]).wait()

        def start_out(s, u):
            t, b0, _ = src_off(s)
            for a in range(4):
                pltpu.async_copy(
                    tr[u].at[a],
                    out_hbm.at[pl.ds(((t * 4 + a) * NB + b0) * 1024,
                                     G * 1024)],
                    smo[u])

        def wait_out(u):
            for a in range(4):
                pltpu.make_async_copy(tr[u].at[a],
                                      out_hbm.at[pl.ds(0, G * 1024)],
                                      smo[u]).wait()

        cols = [jnp.full((16,), c, jnp.int32) for c in range(EMB)]

        def transpose(u):
            @plsc.parallel_loop(0, G * 8, unroll=2)
            def tbody(i):
                bg = i // 8
                l0 = i - bg * 8
                row_idx = bg * 128 + l0 * 16 + iota
                dst_base = bg * 1024 + l0 * 16
                vecs = [plsc.load_gather(rows[u], [row_idx, cols[c]])
                        for c in range(EMB)]
                for a in range(4):
                    for c8 in range(8):
                        tr[u][a, pl.ds(dst_base + c8 * 128, 16)] = \
                            vecs[a * 8 + c8]

        start_idx(0, 0)
        start_idx(1, 1)
        wait_idx(0)
        start_gather(0)

        def body(j, carry):
            for u in (0, 1):
                s = 2 * j + u
                wait_gather(u)

                @pl.when(j >= 1)
                def _():
                    wait_out(u)

                transpose(u)
                start_out(s, u)

                @pl.when(s + 2 < CH_PER_W)
                def _():
                    start_idx(s + 2, u)

                @pl.when(s + 1 < CH_PER_W)
                def _():
                    wait_idx(1 - u)
                    start_gather(1 - u)
            return carry

        lax.fori_loop(0, CH_PER_W // 2, body, 0)
        wait_out(0)
        wait_out(1)

    return gather_kernel


def kernel(tokens, W):
    w_lin = _build_detile()(W.T)
    tf = tokens.T.reshape(S_DIM * T_DIM).astype(jnp.int32)
    flat = _build_gather()(w_lin.reshape(VOCAB, EMB), tf)
    out5 = flat.reshape(T_DIM, 4, NB, 8, 128)
    return out5.transpose(2, 4, 0, 1, 3).reshape(S_DIM, T_DIM, EMB)


# speedup vs baseline: 2.1604x; 1.2413x over previous
import functools

import jax
import jax.numpy as jnp
from jax import lax
from jax.experimental import pallas as pl
from jax.experimental.pallas import tpu as pltpu
from jax.experimental.pallas import tpu_sc as plsc

VOCAB = 1000000
EMB = 32
NW = 32
NCOLS = VOCAB // 128
TAIL = VOCAB - NCOLS * 128

S_DIM = 4096
T_DIM = 200
NB = S_DIM // 128

B_COL = 4
CW = B_COL * 128
N_STEPS = (NCOLS // NW) // B_COL

G = 4
CHUNK_TOK = G * 128
CH_PER_W = T_DIM * (NB // G) // NW


def _make_mesh():
    return plsc.VectorSubcoreMesh(core_axis_name="c", subcore_axis_name="s")


@functools.lru_cache(maxsize=None)
def _build_detile():
    n_cols_w = N_STEPS * B_COL
    rem = NCOLS - n_cols_w * NW

    @functools.partial(
        pl.kernel,
        mesh=_make_mesh(),
        out_type=jax.ShapeDtypeStruct((VOCAB * EMB,), jnp.float32),
        scratch_types=[
            pltpu.VMEM((EMB, CW + 1), jnp.float32),
            pltpu.VMEM((EMB, CW + 1), jnp.float32),
            pltpu.VMEM((CW * EMB,), jnp.float32),
            pltpu.VMEM((CW * EMB,), jnp.float32),
            pltpu.VMEM((TAIL * EMB,), jnp.float32),
            pltpu.SemaphoreType.DMA,
            pltpu.SemaphoreType.DMA,
            pltpu.SemaphoreType.DMA,
            pltpu.SemaphoreType.DMA,
        ],
        compiler_params=pltpu.CompilerParams(use_tc_tiling_on_sc=True,
                                             needs_layout_passes=False),
    )
    def detile_kernel(wt_hbm, out_hbm, wt0, wt1, lin0, lin1, tail_v,
                      si0, si1, so0, so1):
        wid = lax.axis_index("s") * 2 + lax.axis_index("c")
        col0 = wid * n_cols_w
        wt = (wt0, wt1)
        lin = (lin0, lin1)
        si = (si0, si1)
        so = (so0, so1)
        iota = lax.iota(jnp.int32, 16)

        def start_in(step, u):
            q = col0 + step * B_COL
            pltpu.async_copy(wt_hbm.at[:, pl.ds(q * 128, CW)],
                             wt[u].at[:, pl.ds(0, CW)], si[u])

        def wait_in(u):
            pltpu.make_async_copy(wt_hbm.at[:, pl.ds(0, CW)],
                                  wt[u].at[:, pl.ds(0, CW)], si[u]).wait()

        def start_out(step, u):
            q = col0 + step * B_COL
            pltpu.async_copy(lin[u],
                             out_hbm.at[pl.ds(q * 128 * EMB, CW * EMB)],
                             so[u])

        def wait_out(u):
            pltpu.make_async_copy(lin[u],
                                  out_hbm.at[pl.ds(0, CW * EMB)],
                                  so[u]).wait()

        def transpose(u):
            @plsc.parallel_loop(0, CW // 16, unroll=2)
            def tbody(l0):
                base = l0 * 16
                vecs = []
                for li in range(16):
                    lvec = jnp.broadcast_to(base + li, (16,)).astype(jnp.int32)
                    for c0 in (0, 16):
                        vecs.append(
                            plsc.load_gather(wt[u], [c0 + iota, lvec]))
                for li in range(16):
                    for k, c0 in enumerate((0, 16)):
                        lin[u][pl.ds((base + li) * EMB + c0, 16)] = \
                            vecs[li * 2 + k]

        start_in(0, 0)
        start_in(1, 1)

        def body(j, carry):
            for u in (0, 1):
                step = 2 * j + u
                wait_in(u)

                @pl.when(j >= 1)
                def _():
                    wait_out(u)

                transpose(u)
                start_out(step, u)

                @pl.when(step + 2 < N_STEPS)
                def _():
                    start_in(step + 2, u)
            return carry

        lax.fori_loop(0, N_STEPS // 2, body, 0)
        wait_in(0)
        wait_out(0)
        transpose(0)
        start_out(N_STEPS - 1, 0)
        wait_out(1)
        wait_out(0)

        @pl.when(wid < rem)
        def _():
            q = NW * n_cols_w + wid
            pltpu.sync_copy(wt_hbm.at[:, pl.ds(q * 128, 128)],
                            wt0.at[:, pl.ds(0, 128)])

            def tbody(l0, carry):
                base = (l0 * 16 + iota) * EMB
                for c in range(EMB):
                    vec = wt0[c, pl.ds(l0 * 16, 16)]
                    plsc.store_scatter(lin0, [base + c], vec)
                return carry
            lax.fori_loop(0, 8, tbody, 0)
            pltpu.sync_copy(lin0.at[pl.ds(0, 128 * EMB)],
                            out_hbm.at[pl.ds(q * 128 * EMB, 128 * EMB)])

        @pl.when(wid == rem)
        def _():
            base = NCOLS * 128
            for c in range(EMB):
                pltpu.sync_copy(wt_hbm.at[c, pl.ds(base, TAIL)],
                                wt0.at[c, pl.ds(0, TAIL)])
            for l0 in range(TAIL // 16):
                idx = (l0 * 16 + iota) * EMB
                for c in range(EMB):
                    vec = wt0[c, pl.ds(l0 * 16, 16)]
                    plsc.store_scatter(tail_v, [idx + c], vec)
            pltpu.sync_copy(tail_v, out_hbm.at[pl.ds(base * EMB, TAIL * EMB)])

    return detile_kernel


@functools.lru_cache(maxsize=None)
def _build_gather():
    chunks_per_t = NB // G

    @functools.partial(
        pl.kernel,
        mesh=_make_mesh(),
        out_type=jax.ShapeDtypeStruct((S_DIM * T_DIM * EMB // 128, 128),
                                      jnp.float32),
        scratch_types=[
            pltpu.VMEM((CHUNK_TOK,), jnp.int32),
            pltpu.VMEM((CHUNK_TOK,), jnp.int32),
            pltpu.VMEM((CHUNK_TOK, EMB), jnp.float32),
            pltpu.VMEM((CHUNK_TOK, EMB), jnp.float32),
            pltpu.VMEM((EMB, G * 128 + 1), jnp.float32),
            pltpu.VMEM((EMB, G * 128 + 1), jnp.float32),
            pltpu.SemaphoreType.DMA,
            pltpu.SemaphoreType.DMA,
            pltpu.SemaphoreType.DMA,
            pltpu.SemaphoreType.DMA,
            pltpu.SemaphoreType.DMA,
            pltpu.SemaphoreType.DMA,
        ],
        compiler_params=pltpu.CompilerParams(use_tc_tiling_on_sc=False,
                                             needs_layout_passes=False),
    )
    def gather_kernel(table_hbm, idx_hbm, out_hbm,
                      idx0, idx1, rows0, rows1, tr0, tr1,
                      smi0, smi1, smg0, smg1, smo0, smo1):
        wid = lax.axis_index("s") * 2 + lax.axis_index("c")
        k0 = wid * CH_PER_W
        idx = (idx0, idx1)
        rows = (rows0, rows1)
        tr = (tr0, tr1)
        smi = (smi0, smi1)
        smg = (smg0, smg1)
        smo = (smo0, smo1)
        iota = lax.iota(jnp.int32, 16)

        def src_off(s):
            k = k0 + s
            t = k // chunks_per_t
            b0 = (k - t * chunks_per_t) * G
            return t, b0, t * S_DIM + b0 * 128

        def start_idx(s, u):
            _, _, off = src_off(s)
            pltpu.async_copy(idx_hbm.at[pl.ds(off, CHUNK_TOK)], idx[u],
                             smi[u])

        def wait_idx(u):
            pltpu.make_async_copy(idx_hbm.at[pl.ds(0, CHUNK_TOK)], idx[u],
                                  smi[u]).wait()

        def start_gather(u):
            pltpu.async_copy(table_hbm.at[idx[u]], rows[u], smg[u])

        def wait_gather(u):
            pltpu.make_async_copy(table_hbm.at[idx[u]], rows[u],
                                  smg[u]).wait()

        def start_out(s, u):
            t, b0, _ = src_off(s)
            for a in range(4):
                for bg in range(G):
                    pltpu.async_copy(
                        tr[u].at[pl.ds(a * 8, 8), pl.ds(bg * 128, 128)],
                        out_hbm.at[pl.ds(((t * 4 + a) * NB + b0 + bg) * 8, 8),
                                   :],
                        smo[u])

        def wait_out(u):
            for a in range(4):
                for bg in range(G):
                    pltpu.make_async_copy(
                        tr[u].at[pl.ds(a * 8, 8), pl.ds(bg * 128, 128)],
                        out_hbm.at[pl.ds(0, 8), :],
                        smo[u]).wait()

        cvecs = [(c0 + iota) for c0 in (0, 16)]

        def transpose(u):
            @plsc.parallel_loop(0, G * 8, unroll=2)
            def tbody(i):
                bg = i // 8
                l0 = i - bg * 8
                base = bg * 128 + l0 * 16
                vecs = []
                for li in range(16):
                    for k in (0, 1):
                        vecs.append(rows[u][base + li, pl.ds(k * 16, 16)])
                for li in range(16):
                    pos = jnp.broadcast_to(base + li, (16,)).astype(jnp.int32)
                    for k in (0, 1):
                        plsc.store_scatter(tr[u], [cvecs[k], pos],
                                           vecs[li * 2 + k])

        start_idx(0, 0)
        start_idx(1, 1)
        wait_idx(0)
        start_gather(0)

        def body(j, carry):
            for u in (0, 1):
                s = 2 * j + u
                wait_gather(u)

                @pl.when(j >= 1)
                def _():
                    wait_out(u)

                transpose(u)
                start_out(s, u)

                @pl.when(s + 2 < CH_PER_W)
                def _():
                    start_idx(s + 2, u)

                @pl.when(s + 1 < CH_PER_W)
                def _():
                    wait_idx(1 - u)
                    start_gather(1 - u)
            return carry

        lax.fori_loop(0, CH_PER_W // 2, body, 0)
        wait_out(0)
        wait_out(1)

    return gather_kernel


def kernel(tokens, W):
    w_lin = _build_detile()(W.T)
    tf = tokens.T.reshape(S_DIM * T_DIM).astype(jnp.int32)
    flat = _build_gather()(w_lin.reshape(VOCAB, EMB), tf)
    out5 = flat.reshape(T_DIM, 4, NB, 8, 128)
    return out5.transpose(2, 4, 0, 1, 3).reshape(S_DIM, T_DIM, EMB)
